# baseline (speedup 1.0000x reference)
import functools

import jax
import jax.numpy as jnp
from jax import lax
from jax.experimental import pallas as pl
from jax.experimental.pallas import tpu as pltpu
from jax.experimental.pallas import tpu_sc as plsc

N = 10000
E = 320000
D = 128
R = 4
H = 128
G = 256
NODE_ALLY = 0

NP = 10240
ROWS_SUB = NP // 16
CHUNK = 128
NSLOT = 2
CH_SUB = 160
EP = 16 * CH_SUB * CHUNK
DEGW = 16


def _tables_body(x_ref, w_ref, out_ref):
    out_ref[0] = jnp.dot(x_ref[...], w_ref[0],
                         preferred_element_type=jnp.float32)


def _build_tables(x_pad, wall):
    bna = 2560
    return pl.pallas_call(
        _tables_body,
        grid=(8, NP // bna),
        in_specs=[
            pl.BlockSpec((bna, D), lambda j, i: (i, 0)),
            pl.BlockSpec((1, D, D), lambda j, i: (j, 0, 0)),
        ],
        out_specs=pl.BlockSpec((1, bna, D), lambda j, i: (j, i, 0)),
        out_shape=jax.ShapeDtypeStruct((8, NP, D), jnp.float32),
    )(x_pad, wall)


DEG_HALF = CH_SUB // 2


def _gidx_body(src_ref, ety_ref, out_ref):
    out_ref[...] = ety_ref[...] * (2 * NP) + src_ref[...] * 2


def _build_gidx(srcp2, etyp2):
    return pl.pallas_call(
        _gidx_body,
        out_shape=jax.ShapeDtypeStruct((EP // CHUNK, CHUNK), jnp.int32),
    )(srcp2, etyp2)


def _edge_body(hr_hbm, gix_hbm, dst_hbm, z2d_hbm, zdeg_hbm,
               ones_hbm, outw_hbm, outv_hbm, outd_hbm,
               acc, dacc, gix_big, dst_big, idx_bs, rows_bs, ones_b, sems):
    c = lax.axis_index("c")
    s = lax.axis_index("s")
    lo = s * ROWS_SUB
    rows = pl.ds(lo, ROWS_SUB)
    orow = pl.ds(c * NP + lo, ROWS_SUB)
    crow = pl.ds(s * CH_SUB, CH_SUB)

    pltpu.sync_copy(ones_hbm, ones_b)
    pltpu.sync_copy(gix_hbm.at[crow], gix_big)
    pltpu.sync_copy(dst_hbm.at[crow], dst_big)
    pltpu.sync_copy(z2d_hbm.at[rows], acc.at[rows])
    pltpu.sync_copy(zdeg_hbm.at[rows], dacc.at[rows])
    plsc.subcore_barrier()

    def run_phase(poff, with_deg):
        shift = poff + c

        def mkidx(ci, p):
            for j in range(CHUNK // 16):
                sl = pl.ds(j * 16, 16)
                idx_bs[p][sl] = gix_big[ci, sl] + shift

        def gstart(p):
            pltpu.async_copy(hr_hbm.at[idx_bs[p]], rows_bs[p], sems[p])

        def gwait(p):
            pltpu.make_async_copy(hr_hbm.at[idx_bs[p]], rows_bs[p],
                                  sems[p]).wait()

        def scat(ci, p):
            pltpu.sync_copy(rows_bs[p], acc.at[dst_big.at[ci]], add=True)
            if with_deg:
                @pl.when((ci < DEG_HALF) == (c == 0))
                def _():
                    pltpu.sync_copy(ones_b, dacc.at[dst_big.at[ci]],
                                    add=True)

        mkidx(0, 0)
        gstart(0)

        def body(k, carry):
            a = 2 * k
            b = a + 1
            mkidx(b, 1)
            gstart(1)
            gwait(0)
            scat(a, 0)

            @pl.when(b + 1 < CH_SUB)
            def _():
                mkidx(b + 1, 0)
                gstart(0)

            gwait(1)
            scat(b, 1)
            return carry

        lax.fori_loop(0, CH_SUB // 2, body, 0)

    run_phase(0, True)
    plsc.subcore_barrier()
    pltpu.sync_copy(acc.at[rows], outw_hbm.at[orow])
    pltpu.sync_copy(dacc.at[rows], outd_hbm.at[orow])

    pltpu.sync_copy(z2d_hbm.at[rows], acc.at[rows])
    plsc.subcore_barrier()
    run_phase(8 * NP, False)
    plsc.subcore_barrier()
    pltpu.sync_copy(acc.at[rows], outv_hbm.at[orow])


def _edge_pass(hr2, gix2, dstp2):
    mesh = plsc.VectorSubcoreMesh(core_axis_name="c", subcore_axis_name="s")
    z2d = jnp.zeros((NP, D // 2), jnp.float32)
    zdeg = jnp.zeros((NP, DEGW), jnp.float32)
    ones_in = jnp.ones((CHUNK, DEGW), jnp.float32)
    k = pl.kernel(
        _edge_body,
        out_type=[
            jax.ShapeDtypeStruct((2 * NP, D // 2), jnp.float32),
            jax.ShapeDtypeStruct((2 * NP, D // 2), jnp.float32),
            jax.ShapeDtypeStruct((2 * NP, DEGW), jnp.float32),
        ],
        mesh=mesh,
        scratch_types=[
            pltpu.VMEM_SHARED((NP, D // 2), jnp.float32),
            pltpu.VMEM_SHARED((NP, DEGW), jnp.float32),
            pltpu.VMEM((CH_SUB, CHUNK), jnp.int32),
            pltpu.VMEM((CH_SUB, CHUNK), jnp.int32),
            tuple(pltpu.VMEM((CHUNK,), jnp.int32) for _ in range(NSLOT)),
            tuple(pltpu.VMEM((CHUNK, D // 2), jnp.float32)
                  for _ in range(NSLOT)),
            pltpu.VMEM((CHUNK, DEGW), jnp.float32),
            tuple(pltpu.SemaphoreType.DMA for _ in range(NSLOT)),
        ],
        compiler_params=pltpu.CompilerParams(use_tc_tiling_on_sc=False),
    )
    return k(hr2, gix2, dstp2, z2d, zdeg, ones_in)


def _post_body(pw_ref, pv_ref, pd_ref, x_ref, qs_ref, nt_ref, gid_ref,
               wsw_ref, bgw_ref, w1w_ref, b1w_ref, w2w_ref, b2w_ref,
               wsv_ref, bgv_ref, w1v_ref, b1v_ref, w2v_ref, b2v_ref,
               out_ref):
    bn = x_ref.shape[0]
    degc = jnp.maximum(pd_ref[0, :, 0:1] + pd_ref[1, :, 0:1], 1.0)
    xv = x_ref[...]

    def branch(p_ref, ws, bg, w1, b1, w2, b2):
        agg = jnp.concatenate([p_ref[0], p_ref[1]], axis=1) / degc
        h = jnp.maximum(
            agg + jnp.dot(xv, ws[...], preferred_element_type=jnp.float32)
            + bg[...], 0.0)
        a1 = jnp.maximum(
            jnp.dot(h, w1[...], preferred_element_type=jnp.float32)
            + b1[...], 0.0)
        return jnp.dot(a1, w2[...],
                       preferred_element_type=jnp.float32) + b2[...]

    wq = jnp.abs(branch(pw_ref, wsw_ref, bgw_ref, w1w_ref, b1w_ref,
                        w2w_ref, b2w_ref))
    vq = branch(pv_ref, wsv_ref, bgv_ref, w1v_ref, b1v_ref,
                w2v_ref, b2v_ref)
    ally = nt_ref[...] == NODE_ALLY
    contrib = jnp.where(ally, wq * qs_ref[...] + vq, 0.0)
    onehot = (gid_ref[...] ==
              lax.broadcasted_iota(jnp.int32, (bn, G), 1)
              ).astype(jnp.float32)
    part = lax.dot_general(contrib, onehot, (((0,), (0,)), ((), ())),
                           preferred_element_type=jnp.float32)
    @pl.when(pl.program_id(0) == 0)
    def _():
        out_ref[...] = jnp.zeros_like(out_ref)
    out_ref[...] += part


def _post(pw, pv, pd, x_pad, qs2, nt2, gid2, params):
    bnc = 2560
    full = lambda *shape: pl.BlockSpec(shape, lambda i: (0,) * len(shape))
    return pl.pallas_call(
        _post_body,
        grid=(NP // bnc,),
        in_specs=[
            pl.BlockSpec((2, bnc, D // 2), lambda i: (0, i, 0)),
            pl.BlockSpec((2, bnc, D // 2), lambda i: (0, i, 0)),
            pl.BlockSpec((2, bnc, DEGW), lambda i: (0, i, 0)),
            pl.BlockSpec((bnc, D), lambda i: (i, 0)),
            pl.BlockSpec((bnc, 1), lambda i: (i, 0)),
            pl.BlockSpec((bnc, 1), lambda i: (i, 0)),
            pl.BlockSpec((bnc, 1), lambda i: (i, 0)),
            full(D, D), full(1, D), full(D, H), full(1, H), full(H, 1),
            full(1, 1),
            full(D, D), full(1, D), full(D, H), full(1, H), full(H, 1),
            full(1, 1),
        ],
        out_specs=pl.BlockSpec((1, G), lambda i: (0, 0)),
        out_shape=jax.ShapeDtypeStruct((1, G), jnp.float32),
    )(pw, pv, pd, x_pad, qs2, nt2, gid2, *params)


def kernel(node_feature, qs, normalized_score, edge_index, edge_type,
           node_type, graph_ids,
           Wr_w, Ws_w, bg_w, W1_w, b1_w, W2_w, b2_w,
           Wr_v, Ws_v, bg_v, W1_v, b1_v, W2_v, b2_v):
    f32 = jnp.float32
    i32 = jnp.int32

    x_pad = jnp.zeros((NP, D), f32).at[:N].set(node_feature)
    wall = jnp.concatenate([Wr_w, Wr_v], axis=0)

    pad_e = EP - E
    srcp = jnp.concatenate([edge_index[0].astype(i32),
                            jnp.zeros((pad_e,), i32)])
    junk_dst = N + (jnp.arange(pad_e, dtype=i32) % (NP - N))
    dstp = jnp.concatenate([edge_index[1].astype(i32), junk_dst])
    etyp = jnp.concatenate([edge_type.astype(i32), jnp.zeros((pad_e,), i32)])

    hr = _build_tables(x_pad, wall)
    hr2 = hr.reshape(16 * NP, D // 2)

    srcp2 = srcp.reshape(EP // CHUNK, CHUNK)
    etyp2 = etyp.reshape(EP // CHUNK, CHUNK)
    dstp2 = dstp.reshape(EP // CHUNK, CHUNK)
    gix2 = _build_gidx(srcp2, etyp2)
    pw, pv, pd = _edge_pass(hr2, gix2, dstp2)
    pw = pw.reshape(2, NP, D // 2)
    pv = pv.reshape(2, NP, D // 2)
    pd = pd.reshape(2, NP, DEGW)

    qs2 = jnp.zeros((NP, 1), f32).at[:N, 0].set(qs)
    nt2 = jnp.full((NP, 1), 1, i32).at[:N, 0].set(node_type.astype(i32))
    gid2 = jnp.zeros((NP, 1), i32).at[:N, 0].set(graph_ids.astype(i32))

    params = (Ws_w, bg_w.reshape(1, D), W1_w, b1_w.reshape(1, H), W2_w,
              b2_w.reshape(1, 1),
              Ws_v, bg_v.reshape(1, D), W1_v, b1_v.reshape(1, H), W2_v,
              b2_v.reshape(1, 1))
    out = _post(pw, pv, pd, x_pad, qs2, nt2, gid2, params)
    return out.reshape(-1)


# speedup vs baseline: 2.4430x; 2.4430x over previous
import functools

import jax
import jax.numpy as jnp
from jax import lax
from jax.experimental import pallas as pl
from jax.experimental.pallas import tpu as pltpu
from jax.experimental.pallas import tpu_sc as plsc

N = 10000
E = 320000
D = 128
R = 4
H = 128
G = 256
NODE_ALLY = 0

NP = 10240
ROWS_SUB = NP // 16
CHUNK = 128
NSLOT = 2
CH_SUB = 158
EP = 16 * CH_SUB * CHUNK
DEGW = 16


def _tables_body(x_ref, w_ref, out_ref):
    out_ref[0] = jnp.dot(x_ref[...], w_ref[0],
                         preferred_element_type=jnp.float32)


def _build_tables(x_pad, wall):
    bna = 2560
    return pl.pallas_call(
        _tables_body,
        grid=(8, NP // bna),
        in_specs=[
            pl.BlockSpec((bna, D), lambda j, i: (i, 0)),
            pl.BlockSpec((1, D, D), lambda j, i: (j, 0, 0)),
        ],
        out_specs=pl.BlockSpec((1, bna, D), lambda j, i: (j, i, 0)),
        out_shape=jax.ShapeDtypeStruct((8, NP, D), jnp.float32),
    )(x_pad, wall)


DEG_HALF = CH_SUB // 2


def _gidx_body(src_ref, ety_ref, out_ref):
    out_ref[...] = ety_ref[...] * (2 * NP) + src_ref[...] * 2


def _build_gidx(srcp2, etyp2):
    return pl.pallas_call(
        _gidx_body,
        out_shape=jax.ShapeDtypeStruct((EP // CHUNK, CHUNK), jnp.int32),
    )(srcp2, etyp2)


def _edge_body(hr_hbm, gix_hbm, dst_hbm, z2d_hbm, zdeg_hbm,
               ones_hbm, outw_hbm, outv_hbm, outd_hbm,
               acc, dacc, gix_big, dst_big, idx_bs, rows_bs, ones_b, sems):
    c = lax.axis_index("c")
    s = lax.axis_index("s")
    lo = s * ROWS_SUB
    rows = pl.ds(lo, ROWS_SUB)
    orow = pl.ds(c * NP + lo, ROWS_SUB)
    crow = pl.ds(s * CH_SUB, CH_SUB)

    pltpu.sync_copy(ones_hbm, ones_b)
    pltpu.sync_copy(gix_hbm.at[crow], gix_big)
    pltpu.sync_copy(dst_hbm.at[crow], dst_big)
    pltpu.sync_copy(z2d_hbm.at[rows], acc.at[rows])
    pltpu.sync_copy(zdeg_hbm.at[rows], dacc.at[rows])
    plsc.subcore_barrier()

    def run_phase(poff, with_deg):
        shift = poff + c

        def mkidx(ci, p):
            for j in range(CHUNK // 16):
                sl = pl.ds(j * 16, 16)
                idx_bs[p][sl] = gix_big[ci, sl] + shift

        def gstart(p):
            pltpu.async_copy(hr_hbm.at[idx_bs[p]], rows_bs[p], sems[p])

        def gwait(p):
            pltpu.make_async_copy(hr_hbm.at[idx_bs[p]], rows_bs[p],
                                  sems[p]).wait()

        def scat(ci, p):
            pltpu.sync_copy(rows_bs[p], acc.at[dst_big.at[ci]], add=True)
            if with_deg:
                @pl.when((ci < DEG_HALF) == (c == 0))
                def _():
                    pltpu.sync_copy(ones_b, dacc.at[dst_big.at[ci]],
                                    add=True)

        mkidx(0, 0)
        gstart(0)

        def body(k, carry):
            a = 2 * k
            b = a + 1
            mkidx(b, 1)
            gstart(1)
            gwait(0)
            scat(a, 0)

            @pl.when(b + 1 < CH_SUB)
            def _():
                mkidx(b + 1, 0)
                gstart(0)

            gwait(1)
            scat(b, 1)
            return carry

        lax.fori_loop(0, CH_SUB // 2, body, 0)

    run_phase(0, True)
    plsc.subcore_barrier()
    pltpu.sync_copy(acc.at[rows], outw_hbm.at[orow])
    pltpu.sync_copy(dacc.at[rows], outd_hbm.at[orow])

    pltpu.sync_copy(z2d_hbm.at[rows], acc.at[rows])
    plsc.subcore_barrier()
    run_phase(8 * NP, False)
    plsc.subcore_barrier()
    pltpu.sync_copy(acc.at[rows], outv_hbm.at[orow])


def _edge_pass(hr2, gix2, dstp2):
    mesh = plsc.VectorSubcoreMesh(core_axis_name="c", subcore_axis_name="s")
    z2d = jnp.zeros((NP, D // 2), jnp.float32)
    zdeg = jnp.zeros((NP, DEGW), jnp.float32)
    ones_in = jnp.ones((CHUNK, DEGW), jnp.float32)
    k = pl.kernel(
        _edge_body,
        out_type=[
            jax.ShapeDtypeStruct((2 * NP, D // 2), jnp.float32),
            jax.ShapeDtypeStruct((2 * NP, D // 2), jnp.float32),
            jax.ShapeDtypeStruct((2 * NP, DEGW), jnp.float32),
        ],
        mesh=mesh,
        scratch_types=[
            pltpu.VMEM_SHARED((NP, D // 2), jnp.float32),
            pltpu.VMEM_SHARED((NP, DEGW), jnp.float32),
            pltpu.VMEM((CH_SUB, CHUNK), jnp.int32),
            pltpu.VMEM((CH_SUB, CHUNK), jnp.int32),
            tuple(pltpu.VMEM((CHUNK,), jnp.int32) for _ in range(NSLOT)),
            tuple(pltpu.VMEM((CHUNK, D // 2), jnp.float32)
                  for _ in range(NSLOT)),
            pltpu.VMEM((CHUNK, DEGW), jnp.float32),
            tuple(pltpu.SemaphoreType.DMA for _ in range(NSLOT)),
        ],
        compiler_params=pltpu.CompilerParams(use_tc_tiling_on_sc=False),
    )
    return k(hr2, gix2, dstp2, z2d, zdeg, ones_in)


def _post_body(pw_ref, pv_ref, pd_ref, x_ref, qs_ref, nt_ref, gid_ref,
               wsw_ref, bgw_ref, w1w_ref, b1w_ref, w2w_ref, b2w_ref,
               wsv_ref, bgv_ref, w1v_ref, b1v_ref, w2v_ref, b2v_ref,
               out_ref):
    bn = x_ref.shape[0]
    degc = jnp.maximum(pd_ref[0, :, 0:1] + pd_ref[1, :, 0:1], 1.0)
    xv = x_ref[...]

    def branch(p_ref, ws, bg, w1, b1, w2, b2):
        agg = jnp.concatenate([p_ref[0], p_ref[1]], axis=1) / degc
        h = jnp.maximum(
            agg + jnp.dot(xv, ws[...], preferred_element_type=jnp.float32)
            + bg[...], 0.0)
        a1 = jnp.maximum(
            jnp.dot(h, w1[...], preferred_element_type=jnp.float32)
            + b1[...], 0.0)
        return jnp.dot(a1, w2[...],
                       preferred_element_type=jnp.float32) + b2[...]

    wq = jnp.abs(branch(pw_ref, wsw_ref, bgw_ref, w1w_ref, b1w_ref,
                        w2w_ref, b2w_ref))
    vq = branch(pv_ref, wsv_ref, bgv_ref, w1v_ref, b1v_ref,
                w2v_ref, b2v_ref)
    ally = nt_ref[...] == NODE_ALLY
    contrib = jnp.where(ally, wq * qs_ref[...] + vq, 0.0)
    onehot = (gid_ref[...] ==
              lax.broadcasted_iota(jnp.int32, (bn, G), 1)
              ).astype(jnp.float32)
    part = lax.dot_general(contrib, onehot, (((0,), (0,)), ((), ())),
                           preferred_element_type=jnp.float32)
    @pl.when(pl.program_id(0) == 0)
    def _():
        out_ref[...] = jnp.zeros_like(out_ref)
    out_ref[...] += part


def _post(pw, pv, pd, x_pad, qs2, nt2, gid2, params):
    bnc = 2560
    full = lambda *shape: pl.BlockSpec(shape, lambda i: (0,) * len(shape))
    return pl.pallas_call(
        _post_body,
        grid=(NP // bnc,),
        in_specs=[
            pl.BlockSpec((2, bnc, D // 2), lambda i: (0, i, 0)),
            pl.BlockSpec((2, bnc, D // 2), lambda i: (0, i, 0)),
            pl.BlockSpec((2, bnc, DEGW), lambda i: (0, i, 0)),
            pl.BlockSpec((bnc, D), lambda i: (i, 0)),
            pl.BlockSpec((bnc, 1), lambda i: (i, 0)),
            pl.BlockSpec((bnc, 1), lambda i: (i, 0)),
            pl.BlockSpec((bnc, 1), lambda i: (i, 0)),
            full(D, D), full(1, D), full(D, H), full(1, H), full(H, 1),
            full(1, 1),
            full(D, D), full(1, D), full(D, H), full(1, H), full(H, 1),
            full(1, 1),
        ],
        out_specs=pl.BlockSpec((1, G), lambda i: (0, 0)),
        out_shape=jax.ShapeDtypeStruct((1, G), jnp.float32),
    )(pw, pv, pd, x_pad, qs2, nt2, gid2, *params)


def kernel(node_feature, qs, normalized_score, edge_index, edge_type,
           node_type, graph_ids,
           Wr_w, Ws_w, bg_w, W1_w, b1_w, W2_w, b2_w,
           Wr_v, Ws_v, bg_v, W1_v, b1_v, W2_v, b2_v):
    f32 = jnp.float32
    i32 = jnp.int32

    x_pad = jnp.zeros((NP, D), f32).at[:N].set(node_feature)
    wall = jnp.concatenate([Wr_w, Wr_v], axis=0)

    pad_e = EP - E
    junk = jnp.arange(pad_e, dtype=i32)
    srcp = jnp.concatenate([edge_index[0].astype(i32), junk % N])
    dstp = jnp.concatenate([edge_index[1].astype(i32), N + junk % (NP - N)])
    etyp = jnp.concatenate([edge_type.astype(i32), junk % R])

    hr = _build_tables(x_pad, wall)
    hr2 = hr.reshape(16 * NP, D // 2)

    srcp2 = srcp.reshape(EP // CHUNK, CHUNK)
    etyp2 = etyp.reshape(EP // CHUNK, CHUNK)
    dstp2 = dstp.reshape(EP // CHUNK, CHUNK)
    gix2 = _build_gidx(srcp2, etyp2)
    pw, pv, pd = _edge_pass(hr2, gix2, dstp2)
    pw = pw.reshape(2, NP, D // 2)
    pv = pv.reshape(2, NP, D // 2)
    pd = pd.reshape(2, NP, DEGW)

    qs2 = jnp.zeros((NP, 1), f32).at[:N, 0].set(qs)
    nt2 = jnp.full((NP, 1), 1, i32).at[:N, 0].set(node_type.astype(i32))
    gid2 = jnp.zeros((NP, 1), i32).at[:N, 0].set(graph_ids.astype(i32))

    params = (Ws_w, bg_w.reshape(1, D), W1_w, b1_w.reshape(1, H), W2_w,
              b2_w.reshape(1, 1),
              Ws_v, bg_v.reshape(1, D), W1_v, b1_v.reshape(1, H), W2_v,
              b2_v.reshape(1, 1))
    out = _post(pw, pv, pd, x_pad, qs2, nt2, gid2, params)
    return out.reshape(-1)


# speedup vs baseline: 2.9258x; 1.1976x over previous
import functools

import jax
import jax.numpy as jnp
from jax import lax
from jax.experimental import pallas as pl
from jax.experimental.pallas import tpu as pltpu
from jax.experimental.pallas import tpu_sc as plsc

N = 10000
E = 320000
D = 128
R = 4
H = 128
G = 256
NODE_ALLY = 0

NP = 10240
ROWS_SUB = NP // 16
CHUNK = 128
NSLOT = 4
CH_SUB = 160
EP = 16 * CH_SUB * CHUNK
DEGW = 16


def _tables_body(x_ref, w_ref, out_ref):
    out_ref[0] = jnp.dot(x_ref[...], w_ref[0],
                         preferred_element_type=jnp.float32)


def _build_tables(x_pad, wall):
    bna = 2560
    return pl.pallas_call(
        _tables_body,
        grid=(8, NP // bna),
        in_specs=[
            pl.BlockSpec((bna, D), lambda j, i: (i, 0)),
            pl.BlockSpec((1, D, D), lambda j, i: (j, 0, 0)),
        ],
        out_specs=pl.BlockSpec((1, bna, D), lambda j, i: (j, i, 0)),
        out_shape=jax.ShapeDtypeStruct((8, NP, D), jnp.float32),
    )(x_pad, wall)


DEG_HALF = CH_SUB // 2


def _gidx_body(src_ref, ety_ref, out_ref):
    out_ref[...] = ety_ref[...] * (2 * NP) + src_ref[...] * 2


def _build_gidx(srcp2, etyp2):
    return pl.pallas_call(
        _gidx_body,
        out_shape=jax.ShapeDtypeStruct((EP // CHUNK, CHUNK), jnp.int32),
    )(srcp2, etyp2)


def _edge_body(hr_hbm, gix_hbm, dst_hbm, z2d_hbm, zdeg_hbm,
               ones_hbm, outw_hbm, outv_hbm, outd_hbm,
               acc, dacc, gix_big, dst_big, idx_bs, rows_bs, ones_b, sems):
    c = lax.axis_index("c")
    s = lax.axis_index("s")
    lo = s * ROWS_SUB
    rows = pl.ds(lo, ROWS_SUB)
    orow = pl.ds(c * NP + lo, ROWS_SUB)
    crow = pl.ds(s * CH_SUB, CH_SUB)

    pltpu.sync_copy(ones_hbm, ones_b)
    pltpu.sync_copy(gix_hbm.at[crow], gix_big)
    pltpu.sync_copy(dst_hbm.at[crow], dst_big)
    pltpu.sync_copy(z2d_hbm.at[rows], acc.at[rows])
    pltpu.sync_copy(zdeg_hbm.at[rows], dacc.at[rows])
    plsc.subcore_barrier()

    def run_phase(poff, with_deg):
        shift = poff + c

        def mkidx(ci, p):
            for j in range(CHUNK // 16):
                sl = pl.ds(j * 16, 16)
                idx_bs[p][sl] = gix_big[ci, sl] + shift

        def gstart(p):
            pltpu.async_copy(hr_hbm.at[idx_bs[p]], rows_bs[p], sems[p])

        def gwait(p):
            pltpu.make_async_copy(hr_hbm.at[idx_bs[p]], rows_bs[p],
                                  sems[p]).wait()

        def scat(ci, p):
            pltpu.sync_copy(rows_bs[p], acc.at[dst_big.at[ci]], add=True)
            if with_deg:
                @pl.when((ci < DEG_HALF) == (c == 0))
                def _():
                    pltpu.sync_copy(ones_b, dacc.at[dst_big.at[ci]],
                                    add=True)

        for p in range(NSLOT - 1):
            mkidx(p, p)
            gstart(p)

        def visit(ci, p):
            q = (p + NSLOT - 1) % NSLOT

            @pl.when(ci + NSLOT - 1 < CH_SUB)
            def _():
                mkidx(ci + NSLOT - 1, q)
                gstart(q)

            gwait(p)
            scat(ci, p)

        def body(k, carry):
            base = k * NSLOT
            for p in range(NSLOT):
                visit(base + p, p)
            return carry

        lax.fori_loop(0, CH_SUB // NSLOT, body, 0)

    run_phase(0, True)
    plsc.subcore_barrier()
    pltpu.sync_copy(acc.at[rows], outw_hbm.at[orow])
    pltpu.sync_copy(dacc.at[rows], outd_hbm.at[orow])

    pltpu.sync_copy(z2d_hbm.at[rows], acc.at[rows])
    plsc.subcore_barrier()
    run_phase(8 * NP, False)
    plsc.subcore_barrier()
    pltpu.sync_copy(acc.at[rows], outv_hbm.at[orow])


def _edge_pass(hr2, gix2, dstp2):
    mesh = plsc.VectorSubcoreMesh(core_axis_name="c", subcore_axis_name="s")
    z2d = jnp.zeros((NP, D // 2), jnp.float32)
    zdeg = jnp.zeros((NP, DEGW), jnp.float32)
    ones_in = jnp.ones((CHUNK, DEGW), jnp.float32)
    k = pl.kernel(
        _edge_body,
        out_type=[
            jax.ShapeDtypeStruct((2 * NP, D // 2), jnp.float32),
            jax.ShapeDtypeStruct((2 * NP, D // 2), jnp.float32),
            jax.ShapeDtypeStruct((2 * NP, DEGW), jnp.float32),
        ],
        mesh=mesh,
        scratch_types=[
            pltpu.VMEM_SHARED((NP, D // 2), jnp.float32),
            pltpu.VMEM_SHARED((NP, DEGW), jnp.float32),
            pltpu.VMEM((CH_SUB, CHUNK), jnp.int32),
            pltpu.VMEM((CH_SUB, CHUNK), jnp.int32),
            tuple(pltpu.VMEM((CHUNK,), jnp.int32) for _ in range(NSLOT)),
            tuple(pltpu.VMEM((CHUNK, D // 2), jnp.float32)
                  for _ in range(NSLOT)),
            pltpu.VMEM((CHUNK, DEGW), jnp.float32),
            tuple(pltpu.SemaphoreType.DMA for _ in range(NSLOT)),
        ],
        compiler_params=pltpu.CompilerParams(use_tc_tiling_on_sc=False),
    )
    return k(hr2, gix2, dstp2, z2d, zdeg, ones_in)


def _post_body(pw_ref, pv_ref, pd_ref, x_ref, qs_ref, nt_ref, gid_ref,
               wsw_ref, bgw_ref, w1w_ref, b1w_ref, w2w_ref, b2w_ref,
               wsv_ref, bgv_ref, w1v_ref, b1v_ref, w2v_ref, b2v_ref,
               out_ref):
    bn = x_ref.shape[0]
    degc = jnp.maximum(pd_ref[0, :, 0:1] + pd_ref[1, :, 0:1], 1.0)
    xv = x_ref[...]

    def branch(p_ref, ws, bg, w1, b1, w2, b2):
        agg = jnp.concatenate([p_ref[0], p_ref[1]], axis=1) / degc
        h = jnp.maximum(
            agg + jnp.dot(xv, ws[...], preferred_element_type=jnp.float32)
            + bg[...], 0.0)
        a1 = jnp.maximum(
            jnp.dot(h, w1[...], preferred_element_type=jnp.float32)
            + b1[...], 0.0)
        return jnp.dot(a1, w2[...],
                       preferred_element_type=jnp.float32) + b2[...]

    wq = jnp.abs(branch(pw_ref, wsw_ref, bgw_ref, w1w_ref, b1w_ref,
                        w2w_ref, b2w_ref))
    vq = branch(pv_ref, wsv_ref, bgv_ref, w1v_ref, b1v_ref,
                w2v_ref, b2v_ref)
    ally = nt_ref[...] == NODE_ALLY
    contrib = jnp.where(ally, wq * qs_ref[...] + vq, 0.0)
    onehot = (gid_ref[...] ==
              lax.broadcasted_iota(jnp.int32, (bn, G), 1)
              ).astype(jnp.float32)
    part = lax.dot_general(contrib, onehot, (((0,), (0,)), ((), ())),
                           preferred_element_type=jnp.float32)
    @pl.when(pl.program_id(0) == 0)
    def _():
        out_ref[...] = jnp.zeros_like(out_ref)
    out_ref[...] += part


def _post(pw, pv, pd, x_pad, qs2, nt2, gid2, params):
    bnc = 2560
    full = lambda *shape: pl.BlockSpec(shape, lambda i: (0,) * len(shape))
    return pl.pallas_call(
        _post_body,
        grid=(NP // bnc,),
        in_specs=[
            pl.BlockSpec((2, bnc, D // 2), lambda i: (0, i, 0)),
            pl.BlockSpec((2, bnc, D // 2), lambda i: (0, i, 0)),
            pl.BlockSpec((2, bnc, DEGW), lambda i: (0, i, 0)),
            pl.BlockSpec((bnc, D), lambda i: (i, 0)),
            pl.BlockSpec((bnc, 1), lambda i: (i, 0)),
            pl.BlockSpec((bnc, 1), lambda i: (i, 0)),
            pl.BlockSpec((bnc, 1), lambda i: (i, 0)),
            full(D, D), full(1, D), full(D, H), full(1, H), full(H, 1),
            full(1, 1),
            full(D, D), full(1, D), full(D, H), full(1, H), full(H, 1),
            full(1, 1),
        ],
        out_specs=pl.BlockSpec((1, G), lambda i: (0, 0)),
        out_shape=jax.ShapeDtypeStruct((1, G), jnp.float32),
    )(pw, pv, pd, x_pad, qs2, nt2, gid2, *params)


def kernel(node_feature, qs, normalized_score, edge_index, edge_type,
           node_type, graph_ids,
           Wr_w, Ws_w, bg_w, W1_w, b1_w, W2_w, b2_w,
           Wr_v, Ws_v, bg_v, W1_v, b1_v, W2_v, b2_v):
    f32 = jnp.float32
    i32 = jnp.int32

    x_pad = jnp.zeros((NP, D), f32).at[:N].set(node_feature)
    wall = jnp.concatenate([Wr_w, Wr_v], axis=0)

    pad_e = EP - E
    junk = jnp.arange(pad_e, dtype=i32)
    srcp = jnp.concatenate([edge_index[0].astype(i32), junk % N])
    dstp = jnp.concatenate([edge_index[1].astype(i32), N + junk % (NP - N)])
    etyp = jnp.concatenate([edge_type.astype(i32), junk % R])

    hr = _build_tables(x_pad, wall)
    hr2 = hr.reshape(16 * NP, D // 2)

    srcp2 = srcp.reshape(EP // CHUNK, CHUNK)
    etyp2 = etyp.reshape(EP // CHUNK, CHUNK)
    dstp2 = dstp.reshape(EP // CHUNK, CHUNK)
    gix2 = _build_gidx(srcp2, etyp2)
    pw, pv, pd = _edge_pass(hr2, gix2, dstp2)
    pw = pw.reshape(2, NP, D // 2)
    pv = pv.reshape(2, NP, D // 2)
    pd = pd.reshape(2, NP, DEGW)

    qs2 = jnp.zeros((NP, 1), f32).at[:N, 0].set(qs)
    nt2 = jnp.full((NP, 1), 1, i32).at[:N, 0].set(node_type.astype(i32))
    gid2 = jnp.zeros((NP, 1), i32).at[:N, 0].set(graph_ids.astype(i32))

    params = (Ws_w, bg_w.reshape(1, D), W1_w, b1_w.reshape(1, H), W2_w,
              b2_w.reshape(1, 1),
              Ws_v, bg_v.reshape(1, D), W1_v, b1_v.reshape(1, H), W2_v,
              b2_v.reshape(1, 1))
    out = _post(pw, pv, pd, x_pad, qs2, nt2, gid2, params)
    return out.reshape(-1)
